# baseline (speedup 1.0000x reference)
import functools

import jax
import jax.numpy as jnp
from jax import lax
from jax.experimental import pallas as pl
from jax.experimental.pallas import tpu as pltpu
from jax.experimental.pallas import tpu_sc as plsc

EPS = 1e-5
BT_BLK = 64

NC, NS, L = 2, 16, 16
NW = NC * NS
N = 256
D = 128


def _tc_ln_kernel(x_ref, pos_ref, gamma_ref, beta_ref, out_ref):
    h = x_ref[...] + pos_ref[...][None, :, :]
    mean = jnp.mean(h, axis=-1, keepdims=True)
    d = h - mean
    var = jnp.mean(d * d, axis=-1, keepdims=True)
    h_norm = d * jax.lax.rsqrt(var + EPS)
    out_ref[...] = h_norm * gamma_ref[...][None, None, :] + beta_ref[...][None, None, :]


def _tc_ln(xf, pos_embed, gamma, beta, bt_off):
    bt = xf.shape[0] - bt_off
    off_blocks = bt_off // BT_BLK
    return pl.pallas_call(
        _tc_ln_kernel,
        grid=(bt // BT_BLK, 2),
        in_specs=[
            pl.BlockSpec((BT_BLK, N // 2, D), lambda i, j: (off_blocks + i, j, 0)),
            pl.BlockSpec((N // 2, D), lambda i, j: (j, 0)),
            pl.BlockSpec((D,), lambda i, j: (0,)),
            pl.BlockSpec((D,), lambda i, j: (0,)),
        ],
        out_specs=pl.BlockSpec((BT_BLK, N // 2, D), lambda i, j: (i, j, 0)),
        out_shape=jax.ShapeDtypeStruct((bt, N, D), xf.dtype),
        compiler_params=pltpu.CompilerParams(
            dimension_semantics=("parallel", "parallel"),
        ),
    )(xf, pos_embed, gamma, beta)


def _hsum(v, perms):
    for p in perms:
        v = v + v.at[p].get(mode="promise_in_bounds")
    return v


def _sc_row_ln(buf, pos_v, r, poff, g, b, perms):
    h = [buf[r, pl.ds(16 * j, 16)] + pos_v[poff + r, pl.ds(16 * j, 16)] for j in range(8)]
    s = h[0]
    q = h[0] * h[0]
    for j in range(1, 8):
        s = s + h[j]
        q = q + h[j] * h[j]
    mv = _hsum(s, perms) * (1.0 / 128.0)
    var = _hsum(q, perms) * (1.0 / 128.0) - mv * mv
    va = var + EPS
    bits = lax.bitcast_convert_type(va, jnp.int32)
    y = lax.bitcast_convert_type(jnp.int32(0x5F3759DF) - (bits >> 1), jnp.float32)
    for _ in range(3):
        y = y * (1.5 - 0.5 * va * y * y)
    for j in range(8):
        buf[r, pl.ds(16 * j, 16)] = (h[j] - mv) * y * g[j] + b[j]


CH = 128
NBUF = 4


def _sc_body(x_hbm, pos_hbm, gamma_hbm, beta_hbm, out_hbm, buf, pos_v, g_v, b_v, in_sem, out_sem):
    wid = lax.axis_index("s") * NC + lax.axis_index("c")
    per_w = out_hbm.shape[0] // NW
    base = wid * per_w

    pltpu.sync_copy(pos_hbm, pos_v)
    pltpu.sync_copy(gamma_hbm, g_v)
    pltpu.sync_copy(beta_hbm, b_v)

    g = [g_v[pl.ds(16 * j, 16)] for j in range(8)]
    b = [b_v[pl.ds(16 * j, 16)] for j in range(8)]
    iota = lax.iota(jnp.int32, L)
    perms = [iota ^ k for k in (8, 4, 2, 1)]

    pltpu.async_copy(x_hbm.at[base], buf.at[0], in_sem.at[0])
    pltpu.async_copy(x_hbm.at[base + 1], buf.at[1], in_sem.at[1])

    def step_body(t, carry):
        for k in range(NBUF):
            gidx = t * NBUF + k
            idx = base + gidx
            pltpu.make_async_copy(x_hbm.at[idx], buf.at[k], in_sem.at[k]).wait()
            poff = (idx & 1) * CH

            def row_body(r, c):
                _sc_row_ln(buf.at[k], pos_v, r, poff, g, b, perms)
                return c

            lax.fori_loop(0, CH, row_body, 0, unroll=8)
            pltpu.async_copy(buf.at[k], out_hbm.at[idx], out_sem.at[k])

            k2 = (k + 2) % NBUF

            @pl.when(gidx + 2 < per_w)
            def _():
                @pl.when(gidx >= 2)
                def _():
                    pltpu.make_async_copy(
                        buf.at[k2], out_hbm.at[idx - 2], out_sem.at[k2]
                    ).wait()

                pltpu.async_copy(x_hbm.at[idx + 2], buf.at[k2], in_sem.at[k2])

        return carry

    lax.fori_loop(0, per_w // NBUF, step_body, 0)

    pltpu.make_async_copy(buf.at[2], out_hbm.at[base + per_w - 2], out_sem.at[2]).wait()
    pltpu.make_async_copy(buf.at[3], out_hbm.at[base + per_w - 1], out_sem.at[3]).wait()


def _sc_ln(xf, pos_embed, gamma, beta, bt_sc):
    bt = xf.shape[0]
    xc = xf.reshape(bt * (N // CH), CH, D)
    run = pl.kernel(
        _sc_body,
        out_type=jax.ShapeDtypeStruct((bt_sc * (N // CH), CH, D), jnp.float32),
        mesh=plsc.VectorSubcoreMesh(core_axis_name="c", subcore_axis_name="s"),
        scratch_types=[
            pltpu.VMEM((NBUF, CH, D), jnp.float32),
            pltpu.VMEM((N, D), jnp.float32),
            pltpu.VMEM((D,), jnp.float32),
            pltpu.VMEM((D,), jnp.float32),
            pltpu.SemaphoreType.DMA((NBUF,)),
            pltpu.SemaphoreType.DMA((NBUF,)),
        ],
    )
    return run(xc, pos_embed, gamma, beta).reshape(bt_sc, N, D)


def kernel(x, pos_embed, gamma, beta):
    B, T, Nn, Dd = x.shape
    xf = x.reshape(B * T, Nn, Dd)
    out = _tc_ln(xf, pos_embed, gamma, beta, 0)
    return out.reshape(B, T, Nn, Dd)


# speedup vs baseline: 1.1000x; 1.1000x over previous
import jax
import jax.numpy as jnp
from jax.experimental import pallas as pl
from jax.experimental.pallas import tpu as pltpu

EPS = 1e-5
BT_BLK = 64


def _ln_kernel(x_ref, pos_ref, gamma_ref, beta_ref, out_ref):
    h = x_ref[...] + pos_ref[...][None, :, :]
    mean = jnp.mean(h, axis=-1, keepdims=True)
    d = h - mean
    var = jnp.mean(d * d, axis=-1, keepdims=True)
    h_norm = d * jax.lax.rsqrt(var + EPS)
    out_ref[...] = h_norm * gamma_ref[...][None, None, :] + beta_ref[...][None, None, :]


def kernel(x, pos_embed, gamma, beta):
    B, T, N, D = x.shape
    xf = x.reshape(B * T, N, D)
    out = pl.pallas_call(
        _ln_kernel,
        grid=((B * T) // BT_BLK,),
        in_specs=[
            pl.BlockSpec((BT_BLK, N, D), lambda i: (i, 0, 0)),
            pl.BlockSpec((N, D), lambda i: (0, 0)),
            pl.BlockSpec((D,), lambda i: (0,)),
            pl.BlockSpec((D,), lambda i: (0,)),
        ],
        out_specs=pl.BlockSpec((BT_BLK, N, D), lambda i: (i, 0, 0)),
        out_shape=jax.ShapeDtypeStruct((B * T, N, D), x.dtype),
        compiler_params=pltpu.CompilerParams(
            dimension_semantics=("parallel",),
        ),
    )(xf, pos_embed, gamma, beta)
    return out.reshape(B, T, N, D)
